# Initial kernel scaffold; baseline (speedup 1.0000x reference)
import jax
import jax.numpy as jnp
from jax.experimental import pallas as pl


def kernel(input_ids, token_table, position_table):
    raise NotImplementedError("write your pallas kernel here")


# speedup vs baseline: 1.4475x; 1.4475x over previous
import functools

import jax
import jax.numpy as jnp
from jax import lax
from jax.experimental import pallas as pl
from jax.experimental.pallas import tpu as pltpu
from jax.experimental.pallas import tpu_sc as plsc

VOCAB = 1000000
D = 32
L_SEQ = 200
BATCH = 4096
NB = BATCH * L_SEQ
NC, NS = 2, 16
NW = NC * NS
PER_W = NB // NW
IDXW = 128
CHUNK = 3200
GPC = CHUNK // IDXW
ROWS_PER_CHUNK = CHUNK // L_SEQ
NCHUNK = PER_W // CHUNK

_mesh = plsc.VectorSubcoreMesh(
    core_axis_name="c", subcore_axis_name="s", num_cores=NC, num_subcores=NS
)


@functools.partial(
    pl.kernel,
    out_type=jax.ShapeDtypeStruct((NB, D), jnp.float32),
    mesh=_mesh,
    compiler_params=pltpu.CompilerParams(use_tc_tiling_on_sc=False),
    scratch_types=[
        pltpu.VMEM((CHUNK,), jnp.int32),
        pltpu.VMEM((CHUNK, D), jnp.float32),
        pltpu.VMEM((L_SEQ, D), jnp.float32),
        pltpu.SemaphoreType.DMA,
    ],
)
def _emb_kernel(ids_hbm, tok_hbm, pos_hbm, out_hbm, idx_v, rows_v, pos_v, sem):
    wid = lax.axis_index("s") * NC + lax.axis_index("c")
    base = wid * PER_W

    pltpu.sync_copy(pos_hbm, pos_v)

    def chunk_body(c, carry):
        off = base + c * CHUNK
        pltpu.sync_copy(ids_hbm.at[pl.ds(off, CHUNK)], idx_v)
        copies = []
        for j in range(GPC):
            copies.append(
                pltpu.async_copy(
                    tok_hbm.at[idx_v.at[pl.ds(j * IDXW, IDXW)]],
                    rows_v.at[pl.ds(j * IDXW, IDXW)],
                    sem,
                )
            )
        for cp in copies:
            cp.wait()

        def pos_body(l, carry2):
            p0 = pos_v[l, pl.ds(0, 16)]
            p1 = pos_v[l, pl.ds(16, 16)]
            for g in range(ROWS_PER_CHUNK):
                r = g * L_SEQ + l
                rows_v[r, pl.ds(0, 16)] += p0
                rows_v[r, pl.ds(16, 16)] += p1
            return carry2

        lax.fori_loop(0, L_SEQ, pos_body, 0)

        pltpu.sync_copy(rows_v, out_hbm.at[pl.ds(off, CHUNK)])
        return carry

    lax.fori_loop(0, NCHUNK, chunk_body, 0)


def kernel(input_ids, token_table, position_table):
    ids = input_ids.reshape(NB).astype(jnp.int32)
    out = _emb_kernel(ids, token_table, position_table)
    return out.reshape(BATCH, L_SEQ, D)
